# speedup vs baseline: 1.3430x; 1.3430x over previous
import functools

import jax
import jax.numpy as jnp
from jax import lax
from jax.experimental import pallas as pl
from jax.experimental.pallas import tpu as pltpu
from jax.experimental.pallas import tpu_sc as plsc

DIM = 128
NUM_CORES = 2
NUM_SUBCORES = 16
NUM_WORKERS = NUM_CORES * NUM_SUBCORES
IDX_CHUNK = 128


def _emb_body(seq_len, rows_per_worker, ids_hbm, word_hbm, pos_hbm, out_hbm,
              idx_v, rows_v, sem):
    wid = lax.axis_index("s") * NUM_CORES + lax.axis_index("c")
    base = wid * rows_per_worker
    s_base = lax.rem(base, seq_len)
    n_chunks = rows_per_worker // IDX_CHUNK
    pltpu.sync_copy(ids_hbm.at[pl.ds(wid * n_chunks, n_chunks)], idx_v)
    pltpu.sync_copy(pos_hbm.at[pl.ds(s_base, rows_per_worker)], rows_v)
    cps = [
        pltpu.async_copy(
            word_hbm.at[idx_v.at[j]],
            rows_v.at[pl.ds(j * IDX_CHUNK, IDX_CHUNK)],
            sem,
            add=True,
        )
        for j in range(n_chunks)
    ]
    for cp in cps:
        cp.wait()
    pltpu.sync_copy(rows_v, out_hbm.at[pl.ds(base, rows_per_worker)])


def kernel(input_ids, word_embeddings, position_embeddings):
    batch, seq_len = input_ids.shape
    total = batch * seq_len
    rows_per_worker = total // NUM_WORKERS
    ids2d = input_ids.reshape(total // IDX_CHUNK, IDX_CHUNK).astype(jnp.int32)
    mesh = plsc.VectorSubcoreMesh(core_axis_name="c", subcore_axis_name="s")
    body = functools.partial(_emb_body, seq_len, rows_per_worker)
    out = pl.kernel(
        body,
        mesh=mesh,
        out_type=jax.ShapeDtypeStruct((total, DIM), jnp.float32),
        scratch_types=[
            pltpu.VMEM((rows_per_worker // IDX_CHUNK, IDX_CHUNK), jnp.int32),
            pltpu.VMEM((rows_per_worker, DIM), jnp.float32),
            pltpu.SemaphoreType.DMA,
        ],
    )(ids2d, word_embeddings, position_embeddings)
    return out.reshape(batch, seq_len, DIM)


# speedup vs baseline: 1.3711x; 1.0209x over previous
import functools

import jax
import jax.numpy as jnp
from jax import lax
from jax.experimental import pallas as pl
from jax.experimental.pallas import tpu as pltpu
from jax.experimental.pallas import tpu_sc as plsc

DIM = 128
NUM_CORES = 2
NUM_SUBCORES = 16
NUM_WORKERS = NUM_CORES * NUM_SUBCORES
CHUNK = 64


def _emb_body(seq_len, rows_per_worker, ids_hbm, word_hbm, pos_hbm, out_hbm,
              idx_v, rows_v, sem_idx, sem_pos, sem_gat, sem_out):
    n_chunks = rows_per_worker // CHUNK
    wid = lax.axis_index("s") * NUM_CORES + lax.axis_index("c")
    base = wid * rows_per_worker
    s_base = lax.rem(base, seq_len)
    idx_cp = pltpu.async_copy(ids_hbm.at[pl.ds(wid * n_chunks, n_chunks)],
                              idx_v, sem_idx)
    pos_cps = [
        pltpu.async_copy(pos_hbm.at[pl.ds(s_base + j * CHUNK, CHUNK)],
                         rows_v.at[pl.ds(j * CHUNK, CHUNK)], sem_pos.at[j])
        for j in range(n_chunks)
    ]
    idx_cp.wait()
    gat_cps = []
    for j in range(n_chunks):
        pos_cps[j].wait()
        gat_cps.append(pltpu.async_copy(
            word_hbm.at[idx_v.at[j]],
            rows_v.at[pl.ds(j * CHUNK, CHUNK)], sem_gat.at[j], add=True))
    out_cps = []
    for j in range(n_chunks):
        gat_cps[j].wait()
        out_cps.append(pltpu.async_copy(
            rows_v.at[pl.ds(j * CHUNK, CHUNK)],
            out_hbm.at[pl.ds(base + j * CHUNK, CHUNK)], sem_out.at[j]))
    for cp in out_cps:
        cp.wait()


def kernel(input_ids, word_embeddings, position_embeddings):
    batch, seq_len = input_ids.shape
    total = batch * seq_len
    rows_per_worker = total // NUM_WORKERS
    n_chunks = rows_per_worker // CHUNK
    ids2d = input_ids.reshape(total // CHUNK, CHUNK).astype(jnp.int32)
    mesh = plsc.VectorSubcoreMesh(core_axis_name="c", subcore_axis_name="s")
    body = functools.partial(_emb_body, seq_len, rows_per_worker)
    out = pl.kernel(
        body,
        mesh=mesh,
        out_type=jax.ShapeDtypeStruct((total, DIM), jnp.float32),
        scratch_types=[
            pltpu.VMEM((n_chunks, CHUNK), jnp.int32),
            pltpu.VMEM((rows_per_worker, DIM), jnp.float32),
            pltpu.SemaphoreType.DMA,
            pltpu.SemaphoreType.DMA((n_chunks,)),
            pltpu.SemaphoreType.DMA((n_chunks,)),
            pltpu.SemaphoreType.DMA((n_chunks,)),
        ],
    )(ids2d, word_embeddings, position_embeddings)
    return out.reshape(batch, seq_len, DIM)


# speedup vs baseline: 1.3781x; 1.0051x over previous
import functools

import jax
import jax.numpy as jnp
from jax import lax
from jax.experimental import pallas as pl
from jax.experimental.pallas import tpu as pltpu
from jax.experimental.pallas import tpu_sc as plsc

DIM = 128
NUM_CORES = 2
NUM_SUBCORES = 16
NUM_WORKERS = NUM_CORES * NUM_SUBCORES
CHUNK = 64


def _emb_body(seq_len, rows_per_worker, ids_hbm, word_hbm, pos_hbm, out_hbm,
              idx_v, rows_v, sem_idx, sem_pos, sem_gat, sem_out):
    n_chunks = rows_per_worker // CHUNK
    wid = lax.axis_index("s") * NUM_CORES + lax.axis_index("c")
    base = wid * rows_per_worker
    workers_per_batch = seq_len // rows_per_worker
    b = wid // workers_per_batch
    s_base = lax.rem(base, seq_len)
    idx_cp = pltpu.async_copy(ids_hbm.at[b, pl.ds(s_base, rows_per_worker)],
                              idx_v, sem_idx)
    pos_cps = [
        pltpu.async_copy(pos_hbm.at[pl.ds(s_base + j * CHUNK, CHUNK)],
                         rows_v.at[pl.ds(j * CHUNK, CHUNK)], sem_pos.at[j])
        for j in range(n_chunks)
    ]
    idx_cp.wait()
    gat_cps = []
    for j in range(n_chunks):
        pos_cps[j].wait()
        gat_cps.append(pltpu.async_copy(
            word_hbm.at[idx_v.at[pl.ds(j * CHUNK, CHUNK)]],
            rows_v.at[pl.ds(j * CHUNK, CHUNK)], sem_gat.at[j], add=True))
    out_cps = []
    for j in range(n_chunks):
        gat_cps[j].wait()
        out_cps.append(pltpu.async_copy(
            rows_v.at[pl.ds(j * CHUNK, CHUNK)],
            out_hbm.at[pl.ds(base + j * CHUNK, CHUNK)], sem_out.at[j]))
    for cp in out_cps:
        cp.wait()


def kernel(input_ids, word_embeddings, position_embeddings):
    batch, seq_len = input_ids.shape
    total = batch * seq_len
    rows_per_worker = total // NUM_WORKERS
    n_chunks = rows_per_worker // CHUNK
    mesh = plsc.VectorSubcoreMesh(core_axis_name="c", subcore_axis_name="s")
    body = functools.partial(_emb_body, seq_len, rows_per_worker)
    out = pl.kernel(
        body,
        mesh=mesh,
        out_type=jax.ShapeDtypeStruct((total, DIM), jnp.float32),
        scratch_types=[
            pltpu.VMEM((rows_per_worker,), jnp.int32),
            pltpu.VMEM((rows_per_worker, DIM), jnp.float32),
            pltpu.SemaphoreType.DMA,
            pltpu.SemaphoreType.DMA((n_chunks,)),
            pltpu.SemaphoreType.DMA((n_chunks,)),
            pltpu.SemaphoreType.DMA((n_chunks,)),
        ],
    )(input_ids, word_embeddings, position_embeddings)
    return out.reshape(batch, seq_len, DIM)
